# baseline (speedup 1.0000x reference)
import functools

import jax
import jax.numpy as jnp
import numpy as np
from jax import lax
from jax.experimental import pallas as pl
from jax.experimental.pallas import tpu as pltpu
from jax.experimental.pallas import tpu_sc as plsc

_ROWS = 128
_COLS = 32768
_TC_ROWS = 64
_SC_BASE_PACKED = _TC_ROWS // 2
_NUM_CORES = 2
_NUM_SUBCORES = 16
_NUM_WORKERS = _NUM_CORES * _NUM_SUBCORES
_LANES = 16
_HALF = _COLS // 2
_CHUNK_ITERS = _HALF // _LANES
_ROW_ITERS = 2 * _CHUNK_ITERS

_SIGN2 = np.int32(-2147450880)
_HI16 = np.int32(-65536)
_ONE2 = np.int32(0x00010001)
_INT32_MIN = np.int32(-(2**31))


def _scan_chunk(buf, iter0, carry):
    def body(i, carry):
        acc_l, acc_h, inv = carry
        v = buf[pl.ds(i * _LANES, _LANES)]
        acc_l = jnp.maximum(acc_l, (v << 16) | inv)
        acc_h = jnp.maximum(acc_h, (v & _HI16) | inv)
        return acc_l, acc_h, inv - 1

    inv0 = jnp.full((_LANES,), np.int32(_ROW_ITERS - 1 - iter0), jnp.int32)
    acc_l, acc_h, _ = lax.fori_loop(
        0, _CHUNK_ITERS, body, (*carry, inv0), unroll=8
    )
    return acc_l, acc_h


def _scan_row_exact(rows_v):
    def body(i, carry):
        acc_l, acc_h, inv = carry
        v = rows_v[pl.ds(i * _LANES, _LANES)]
        m = (v & _SIGN2) - ((v >> 15) & _ONE2)
        kk = v ^ m
        acc_l = jnp.maximum(acc_l, (kk << 16) | inv)
        acc_h = jnp.maximum(acc_h, (kk & _HI16) | inv)
        return acc_l, acc_h, inv - 1

    inv0 = jnp.full((_LANES,), np.int32(_ROW_ITERS - 1), jnp.int32)
    acc0 = jnp.full((_LANES,), _INT32_MIN, jnp.int32)
    acc_l, acc_h, _ = lax.fori_loop(0, _ROW_ITERS, body, (acc0, acc0, inv0))
    return acc_l, acc_h


def _merge_lanes(acc):
    best = acc[0]
    lane = np.int32(0)
    for j in range(1, _LANES):
        a = acc[j]
        upd = a > best
        best = jnp.where(upd, a, best)
        lane = jnp.where(upd, np.int32(j), lane)
    return best, lane


def _to_col(best, lane):
    it = np.int32(_ROW_ITERS - 1) - (best & np.int32(0xFFFF))
    return it * _LANES + lane


def _finish_row(rows_v, acc_l, acc_h):
    best_l, lane_l = _merge_lanes(acc_l)
    best_h, lane_h = _merge_lanes(acc_h)
    ok = jnp.logical_and(best_l > np.int32(65535), best_h > np.int32(65535))

    def fast(_):
        return _to_col(best_l, lane_l), _to_col(best_h, lane_h)

    def exact(_):
        a_l, a_h = _scan_row_exact(rows_v)
        b_l, ln_l = _merge_lanes(a_l)
        b_h, ln_h = _merge_lanes(a_h)
        return _to_col(b_l, ln_l), _to_col(b_h, ln_h)

    return lax.cond(ok, fast, exact, None)


@functools.partial(
    pl.kernel,
    mesh=plsc.VectorSubcoreMesh(core_axis_name="c", subcore_axis_name="s"),
    out_type=jax.ShapeDtypeStruct((_NUM_WORKERS, 16), jnp.int32),
    scratch_types=[
        pltpu.VMEM((_COLS,), jnp.int32),
        pltpu.VMEM((16,), jnp.int32),
        pltpu.SemaphoreType.DMA,
        pltpu.SemaphoreType.DMA,
    ],
)
def _sc_argmax(x_hbm, out_hbm, row_v, res_v, sem0, sem1):
    wid = lax.axis_index("s") * _NUM_CORES + lax.axis_index("c")
    x32 = x_hbm.bitcast(jnp.int32)
    p = _SC_BASE_PACKED + wid
    cp0 = pltpu.async_copy(
        x32.at[p, pl.ds(0, _HALF)], row_v.at[pl.ds(0, _HALF)], sem0
    )
    cp1 = pltpu.async_copy(
        x32.at[p, pl.ds(_HALF, _HALF)], row_v.at[pl.ds(_HALF, _HALF)], sem1
    )
    acc0 = jnp.full((_LANES,), _INT32_MIN, jnp.int32)
    cp0.wait()
    acc = _scan_chunk(row_v.at[pl.ds(0, _HALF)], 0, (acc0, acc0))
    cp1.wait()
    acc = _scan_chunk(row_v.at[pl.ds(_HALF, _HALF)], _CHUNK_ITERS, acc)
    idx_l, idx_h = _finish_row(row_v, *acc)
    lane16 = lax.iota(jnp.int32, 16)
    res = jnp.where(lane16 == 0, idx_l, jnp.zeros((16,), jnp.int32))
    res = jnp.where(lane16 == 1, idx_h, res)
    res_v[...] = res
    pltpu.sync_copy(res_v, out_hbm.at[wid])


def _tc_key(x):
    return x ^ (jnp.right_shift(x, np.int32(15)) & np.int32(0x7FFF))


def _tc_body(x_ref, o_ref):
    w = x_ref[...]
    klo = _tc_key(jnp.right_shift(w << 16, np.int32(16)))
    khi = _tc_key(jnp.right_shift(w, np.int32(16)))
    m = jnp.maximum(
        jnp.max(klo, axis=1, keepdims=True),
        jnp.max(khi, axis=1, keepdims=True),
    )
    j2 = 2 * lax.broadcasted_iota(jnp.int32, w.shape, 1)
    big = np.int32(2**30)
    cand = jnp.minimum(
        jnp.min(jnp.where(klo == m, j2, big), axis=1, keepdims=True),
        jnp.min(jnp.where(khi == m, j2 + 1, big), axis=1, keepdims=True),
    )
    o_ref[...] = jnp.broadcast_to(cand, (16, 128))


_tc_argmax = pl.pallas_call(
    _tc_body,
    grid=(_TC_ROWS // 16,),
    in_specs=[pl.BlockSpec((16, _COLS // 2), lambda i: (i, 0))],
    out_specs=pl.BlockSpec((16, 128), lambda i: (i, 0)),
    out_shape=jax.ShapeDtypeStruct((_TC_ROWS, 128), jnp.int32),
)


def kernel(input_tensor, dim):
    del dim
    sc_out = _sc_argmax(input_tensor)
    packed = lax.bitcast_convert_type(
        input_tensor[:_TC_ROWS].reshape(_TC_ROWS, _COLS // 2, 2), jnp.int32
    )
    tc_out = _tc_argmax(packed)[:, 0]
    sc_idx = sc_out[:, :2].reshape(_ROWS - _TC_ROWS)
    return jnp.concatenate([tc_out, sc_idx]).astype(jnp.int64)


# speedup vs baseline: 4.1770x; 4.1770x over previous
import functools

import jax
import jax.numpy as jnp
import numpy as np
from jax import lax
from jax.experimental import pallas as pl
from jax.experimental.pallas import tpu as pltpu
from jax.experimental.pallas import tpu_sc as plsc

_ROWS = 128
_COLS = 32768
_NUM_CORES = 2
_NUM_SUBCORES = 16
_NUM_WORKERS = _NUM_CORES * _NUM_SUBCORES
_PACKED_PER_WORKER = 2
_LANES = 16
_HALF = _COLS // 2
_CHUNK_ITERS = _HALF // _LANES
_ROW_ITERS = 2 * _CHUNK_ITERS

_SIGN2 = np.int32(-2147450880)
_HI16 = np.int32(-65536)
_ONE2 = np.int32(0x00010001)
_INT32_MIN = np.int32(-(2**31))


def _scan_chunk(buf, iter0, n_iters, carry):
    def body(i, carry):
        acc_l, acc_h, inv = carry
        v = buf[pl.ds(i * _LANES, _LANES)]
        acc_l = jnp.maximum(acc_l, (v << 16) | inv)
        acc_h = jnp.maximum(acc_h, (v & _HI16) | inv)
        return acc_l, acc_h, inv - 1

    inv0 = jnp.full((_LANES,), np.int32(_ROW_ITERS - 1 - iter0), jnp.int32)
    acc_l, acc_h, _ = lax.fori_loop(
        0, n_iters, body, (*carry, inv0), unroll=8
    )
    return acc_l, acc_h


def _scan_row_exact(rows_v, rr):
    def body(i, carry):
        acc_l, acc_h, inv = carry
        v = rows_v[rr, pl.ds(i * _LANES, _LANES)]
        m = (v & _SIGN2) - ((v >> 15) & _ONE2)
        kk = v ^ m
        acc_l = jnp.maximum(acc_l, (kk << 16) | inv)
        acc_h = jnp.maximum(acc_h, (kk & _HI16) | inv)
        return acc_l, acc_h, inv - 1

    inv0 = jnp.full((_LANES,), np.int32(_ROW_ITERS - 1), jnp.int32)
    acc0 = jnp.full((_LANES,), _INT32_MIN, jnp.int32)
    acc_l, acc_h, _ = lax.fori_loop(
        0, _ROW_ITERS, body, (acc0, acc0, inv0)
    )
    return acc_l, acc_h


def _merge_lanes(acc):
    best = acc[0]
    lane = np.int32(0)
    for j in range(1, _LANES):
        a = acc[j]
        upd = a > best
        best = jnp.where(upd, a, best)
        lane = jnp.where(upd, np.int32(j), lane)
    return best, lane


def _to_col(best, lane):
    it = np.int32(_ROW_ITERS - 1) - (best & np.int32(0xFFFF))
    return it * _LANES + lane


def _finish_row(rows_v, rr, acc_l, acc_h):
    best_l, lane_l = _merge_lanes(acc_l)
    best_h, lane_h = _merge_lanes(acc_h)
    ok = jnp.logical_and(best_l > np.int32(65535), best_h > np.int32(65535))

    def fast(_):
        return _to_col(best_l, lane_l), _to_col(best_h, lane_h)

    def exact(_):
        a_l, a_h = _scan_row_exact(rows_v, rr)
        b_l, ln_l = _merge_lanes(a_l)
        b_h, ln_h = _merge_lanes(a_h)
        return _to_col(b_l, ln_l), _to_col(b_h, ln_h)

    return lax.cond(ok, fast, exact, None)


@functools.partial(
    pl.kernel,
    mesh=plsc.VectorSubcoreMesh(core_axis_name="c", subcore_axis_name="s"),
    out_type=jax.ShapeDtypeStruct((_NUM_WORKERS, 16), jnp.int32),
    scratch_types=[
        pltpu.VMEM((_PACKED_PER_WORKER, _COLS), jnp.int32),
        pltpu.VMEM((16,), jnp.int32),
    ]
    + [pltpu.SemaphoreType.DMA] * 8,
)
def _sc_argmax(x_hbm, out_hbm, rows_v, res_v, *sems):
    wid = lax.axis_index("s") * _NUM_CORES + lax.axis_index("c")
    x32 = x_hbm.bitcast(jnp.int32)
    base = wid * _PACKED_PER_WORKER
    q = _HALF // 2
    copies = []
    for c in range(8):
        rr, hh = divmod(c, 4)
        copies.append(
            pltpu.async_copy(
                x32.at[base + rr, pl.ds(hh * q, q)],
                rows_v.at[rr, pl.ds(hh * q, q)],
                sems[c],
            )
        )
    lane16 = lax.iota(jnp.int32, 16)
    res = jnp.zeros((16,), jnp.int32)
    acc0 = jnp.full((_LANES,), _INT32_MIN, jnp.int32)
    for rr in range(_PACKED_PER_WORKER):
        acc = (acc0, acc0)
        for hh in range(4):
            copies[4 * rr + hh].wait()
            acc = _scan_chunk(
                rows_v.at[rr, pl.ds(hh * (_HALF // 2), _HALF // 2)],
                hh * (_CHUNK_ITERS // 2),
                _CHUNK_ITERS // 2,
                acc,
            )
        idx_l, idx_h = _finish_row(rows_v, rr, *acc)
        res = jnp.where(lane16 == 2 * rr, idx_l, res)
        res = jnp.where(lane16 == 2 * rr + 1, idx_h, res)
    res_v[...] = res
    pltpu.sync_copy(res_v, out_hbm.at[wid])


def kernel(input_tensor, dim):
    del dim
    out = _sc_argmax(input_tensor)
    return out[:, : 2 * _PACKED_PER_WORKER].reshape(_ROWS).astype(jnp.int64)


# speedup vs baseline: 4.2736x; 1.0231x over previous
import functools

import jax
import jax.numpy as jnp
import numpy as np
from jax import lax
from jax.experimental import pallas as pl
from jax.experimental.pallas import tpu as pltpu
from jax.experimental.pallas import tpu_sc as plsc

_ROWS = 128
_COLS = 32768
_NUM_CORES = 2
_NUM_SUBCORES = 16
_NUM_WORKERS = _NUM_CORES * _NUM_SUBCORES
_PACKED_PER_WORKER = 2
_LANES = 16
_HALF = _COLS // 2
_CHUNK_ITERS = _HALF // _LANES
_ROW_ITERS = 2 * _CHUNK_ITERS

_SIGN2 = np.int32(-2147450880)
_HI16 = np.int32(-65536)
_ONE2 = np.int32(0x00010001)
_INT32_MIN = np.int32(-(2**31))


def _scan_chunk(buf, iter0, carry):
    def body(i, carry):
        acc_l, acc_h, inv = carry
        v = buf[pl.ds(i * _LANES, _LANES)]
        acc_l = jnp.maximum(acc_l, (v << 16) | inv)
        acc_h = jnp.maximum(acc_h, (v & _HI16) | inv)
        return acc_l, acc_h, inv - 1

    inv0 = jnp.full((_LANES,), np.int32(_ROW_ITERS - 1 - iter0), jnp.int32)
    acc_l, acc_h, _ = lax.fori_loop(
        0, _CHUNK_ITERS, body, (*carry, inv0), unroll=8
    )
    return acc_l, acc_h


def _scan_row_exact(rows_v, rr):
    def body(i, carry):
        acc_l, acc_h, inv = carry
        v = rows_v[rr, pl.ds(i * _LANES, _LANES)]
        m = (v & _SIGN2) - ((v >> 15) & _ONE2)
        kk = v ^ m
        acc_l = jnp.maximum(acc_l, (kk << 16) | inv)
        acc_h = jnp.maximum(acc_h, (kk & _HI16) | inv)
        return acc_l, acc_h, inv - 1

    inv0 = jnp.full((_LANES,), np.int32(_ROW_ITERS - 1), jnp.int32)
    acc0 = jnp.full((_LANES,), _INT32_MIN, jnp.int32)
    acc_l, acc_h, _ = lax.fori_loop(
        0, _ROW_ITERS, body, (acc0, acc0, inv0)
    )
    return acc_l, acc_h


def _merge_lanes(acc):
    best = acc[0]
    lane = np.int32(0)
    for j in range(1, _LANES):
        a = acc[j]
        upd = a > best
        best = jnp.where(upd, a, best)
        lane = jnp.where(upd, np.int32(j), lane)
    return best, lane


def _to_col(best, lane):
    it = np.int32(_ROW_ITERS - 1) - (best & np.int32(0xFFFF))
    return it * _LANES + lane


def _finish_row(rows_v, rr, acc_l, acc_h):
    best_l, lane_l = _merge_lanes(acc_l)
    best_h, lane_h = _merge_lanes(acc_h)
    ok = jnp.logical_and(best_l > np.int32(65535), best_h > np.int32(65535))

    def fast(_):
        return _to_col(best_l, lane_l), _to_col(best_h, lane_h)

    def exact(_):
        a_l, a_h = _scan_row_exact(rows_v, rr)
        b_l, ln_l = _merge_lanes(a_l)
        b_h, ln_h = _merge_lanes(a_h)
        return _to_col(b_l, ln_l), _to_col(b_h, ln_h)

    return lax.cond(ok, fast, exact, None)


@functools.partial(
    pl.kernel,
    mesh=plsc.VectorSubcoreMesh(core_axis_name="c", subcore_axis_name="s"),
    out_type=jax.ShapeDtypeStruct((_NUM_WORKERS, 16), jnp.int32),
    scratch_types=[
        pltpu.VMEM((_PACKED_PER_WORKER, _COLS), jnp.int32),
        pltpu.VMEM((16,), jnp.int32),
        pltpu.SemaphoreType.DMA,
        pltpu.SemaphoreType.DMA,
        pltpu.SemaphoreType.DMA,
        pltpu.SemaphoreType.DMA,
    ],
)
def _sc_argmax(x_hbm, out_hbm, rows_v, res_v, sem0, sem1, sem2, sem3):
    wid = lax.axis_index("s") * _NUM_CORES + lax.axis_index("c")
    x32 = x_hbm.bitcast(jnp.int32)
    base = wid * _PACKED_PER_WORKER
    sems = (sem0, sem1, sem2, sem3)
    copies = []
    for c in range(4):
        rr, hh = divmod(c, 2)
        copies.append(
            pltpu.async_copy(
                x32.at[base + rr, pl.ds(hh * _HALF, _HALF)],
                rows_v.at[rr, pl.ds(hh * _HALF, _HALF)],
                sems[c],
            )
        )
    lane16 = lax.iota(jnp.int32, 16)
    res = jnp.zeros((16,), jnp.int32)
    acc0 = jnp.full((_LANES,), _INT32_MIN, jnp.int32)
    for rr in range(_PACKED_PER_WORKER):
        copies[2 * rr].wait()
        acc = _scan_chunk(rows_v.at[rr, pl.ds(0, _HALF)], 0, (acc0, acc0))
        copies[2 * rr + 1].wait()
        acc = _scan_chunk(
            rows_v.at[rr, pl.ds(_HALF, _HALF)], _CHUNK_ITERS, acc
        )
        idx_l, idx_h = _finish_row(rows_v, rr, *acc)
        res = jnp.where(lane16 == 2 * rr, idx_l, res)
        res = jnp.where(lane16 == 2 * rr + 1, idx_h, res)
    res_v[...] = res
    pltpu.sync_copy(res_v, out_hbm.at[wid])


def kernel(input_tensor, dim):
    del dim
    out = _sc_argmax(input_tensor)
    return out[:, : 2 * _PACKED_PER_WORKER].reshape(_ROWS).astype(jnp.int64)


# speedup vs baseline: 4.2825x; 1.0021x over previous
import functools

import jax
import jax.numpy as jnp
import numpy as np
from jax import lax
from jax.experimental import pallas as pl
from jax.experimental.pallas import tpu as pltpu
from jax.experimental.pallas import tpu_sc as plsc

_ROWS = 128
_COLS = 32768
_NUM_CORES = 2
_NUM_SUBCORES = 16
_NUM_WORKERS = _NUM_CORES * _NUM_SUBCORES
_PACKED_PER_WORKER = 2
_LANES = 16
_HALF = _COLS // 2
_CHUNK_ITERS = _HALF // _LANES
_ROW_ITERS = 2 * _CHUNK_ITERS

_SIGN2 = np.int32(-2147450880)
_HI16 = np.int32(-65536)
_ONE2 = np.int32(0x00010001)
_INT32_MIN = np.int32(-(2**31))


def _scan_chunk(buf, iter0, carry):
    def body(i, carry):
        acc_l, acc_h, inv = carry
        v = buf[pl.ds(i * _LANES, _LANES)]
        acc_l = jnp.maximum(acc_l, (v << 16) | inv)
        acc_h = jnp.maximum(acc_h, (v & _HI16) | inv)
        return acc_l, acc_h, inv - 1

    inv0 = jnp.full((_LANES,), np.int32(_ROW_ITERS - 1 - iter0), jnp.int32)
    acc_l, acc_h, _ = lax.fori_loop(
        0, _CHUNK_ITERS, body, (*carry, inv0), unroll=8
    )
    return acc_l, acc_h


def _scan_chunk2(rows_v, rr):
    def body(i, carry):
        al0, ah0, inv0_, al1, ah1, inv1_ = carry
        v0 = rows_v[rr, pl.ds(i * _LANES, _LANES)]
        v1 = rows_v[rr, pl.ds(_HALF + i * _LANES, _LANES)]
        al0 = jnp.maximum(al0, (v0 << 16) | inv0_)
        ah0 = jnp.maximum(ah0, (v0 & _HI16) | inv0_)
        al1 = jnp.maximum(al1, (v1 << 16) | inv1_)
        ah1 = jnp.maximum(ah1, (v1 & _HI16) | inv1_)
        return al0, ah0, inv0_ - 1, al1, ah1, inv1_ - 1

    acc0 = jnp.full((_LANES,), _INT32_MIN, jnp.int32)
    ia = jnp.full((_LANES,), np.int32(_ROW_ITERS - 1), jnp.int32)
    ib = jnp.full((_LANES,), np.int32(_CHUNK_ITERS - 1), jnp.int32)
    al0, ah0, _, al1, ah1, _ = lax.fori_loop(
        0, _CHUNK_ITERS, body, (acc0, acc0, ia, acc0, acc0, ib), unroll=4
    )
    return (al0, ah0), (al1, ah1)


def _scan_row_exact(rows_v, rr):
    def body(i, carry):
        acc_l, acc_h, inv = carry
        v = rows_v[rr, pl.ds(i * _LANES, _LANES)]
        m = (v & _SIGN2) - ((v >> 15) & _ONE2)
        kk = v ^ m
        acc_l = jnp.maximum(acc_l, (kk << 16) | inv)
        acc_h = jnp.maximum(acc_h, (kk & _HI16) | inv)
        return acc_l, acc_h, inv - 1

    inv0 = jnp.full((_LANES,), np.int32(_ROW_ITERS - 1), jnp.int32)
    acc0 = jnp.full((_LANES,), _INT32_MIN, jnp.int32)
    acc_l, acc_h, _ = lax.fori_loop(
        0, _ROW_ITERS, body, (acc0, acc0, inv0)
    )
    return acc_l, acc_h


def _merge_lanes(acc):
    best = acc[0]
    lane = np.int32(0)
    for j in range(1, _LANES):
        a = acc[j]
        upd = a > best
        best = jnp.where(upd, a, best)
        lane = jnp.where(upd, np.int32(j), lane)
    return best, lane


def _to_col(best, lane):
    it = np.int32(_ROW_ITERS - 1) - (best & np.int32(0xFFFF))
    return it * _LANES + lane


def _finish_row(rows_v, rr, acc_l, acc_h):
    best_l, lane_l = _merge_lanes(acc_l)
    best_h, lane_h = _merge_lanes(acc_h)
    ok = jnp.logical_and(best_l > np.int32(65535), best_h > np.int32(65535))

    def fast(_):
        return _to_col(best_l, lane_l), _to_col(best_h, lane_h)

    def exact(_):
        a_l, a_h = _scan_row_exact(rows_v, rr)
        b_l, ln_l = _merge_lanes(a_l)
        b_h, ln_h = _merge_lanes(a_h)
        return _to_col(b_l, ln_l), _to_col(b_h, ln_h)

    return lax.cond(ok, fast, exact, None)


@functools.partial(
    pl.kernel,
    mesh=plsc.VectorSubcoreMesh(core_axis_name="c", subcore_axis_name="s"),
    out_type=jax.ShapeDtypeStruct((_NUM_WORKERS, 16), jnp.int32),
    scratch_types=[
        pltpu.VMEM((_PACKED_PER_WORKER, _COLS), jnp.int32),
        pltpu.VMEM((16,), jnp.int32),
        pltpu.SemaphoreType.DMA,
        pltpu.SemaphoreType.DMA,
        pltpu.SemaphoreType.DMA,
        pltpu.SemaphoreType.DMA,
    ],
)
def _sc_argmax(x_hbm, out_hbm, rows_v, res_v, sem0, sem1, sem2, sem3):
    wid = lax.axis_index("s") * _NUM_CORES + lax.axis_index("c")
    x32 = x_hbm.bitcast(jnp.int32)
    base = wid * _PACKED_PER_WORKER
    sems = (sem0, sem1, sem2, sem3)
    copies = []
    for c in range(4):
        rr, hh = divmod(c, 2)
        copies.append(
            pltpu.async_copy(
                x32.at[base + rr, pl.ds(hh * _HALF, _HALF)],
                rows_v.at[rr, pl.ds(hh * _HALF, _HALF)],
                sems[c],
            )
        )
    lane16 = lax.iota(jnp.int32, 16)
    res = jnp.zeros((16,), jnp.int32)
    acc0 = jnp.full((_LANES,), _INT32_MIN, jnp.int32)
    for rr in range(_PACKED_PER_WORKER):
        copies[2 * rr].wait()
        copies[2 * rr + 1].wait()
        a0, a1 = _scan_chunk2(rows_v, rr)
        acc = (jnp.maximum(a0[0], a1[0]), jnp.maximum(a0[1], a1[1]))
        idx_l, idx_h = _finish_row(rows_v, rr, *acc)
        res = jnp.where(lane16 == 2 * rr, idx_l, res)
        res = jnp.where(lane16 == 2 * rr + 1, idx_h, res)
    res_v[...] = res
    pltpu.sync_copy(res_v, out_hbm.at[wid])


def kernel(input_tensor, dim):
    del dim
    out = _sc_argmax(input_tensor)
    return out[:, : 2 * _PACKED_PER_WORKER].reshape(_ROWS).astype(jnp.int64)
